# baseline (speedup 1.0000x reference)
import functools

import jax
import jax.numpy as jnp
from jax.experimental import pallas as pl
from jax.experimental.pallas import tpu as pltpu
from jax.experimental.pallas import tpu_sc as plsc

_N = 2048
_D = 768
_KB = 512
_GW = 128
_SPLIT = 2


def _dist_argmin_body(r_ref, cb_ref, a2_ref, b2_ref, idx_ref, best_ref):
    k = pl.program_id(1)
    ab = jax.lax.dot_general(
        r_ref[...], cb_ref[...],
        dimension_numbers=(((1,), (1,)), ((), ())),
        preferred_element_type=jnp.float32,
    )
    s = a2_ref[...] + b2_ref[...]
    d2 = jnp.maximum(s - 2.0 * ab, 0.0)
    m = jnp.min(d2, axis=1, keepdims=True)
    j = jax.lax.broadcasted_iota(jnp.int32, d2.shape, 1)
    lidx = jnp.min(jnp.where(d2 == m, j, jnp.int32(2**30)), axis=1, keepdims=True)
    gidx = lidx + k * _KB

    @pl.when(k == 0)
    def _():
        best_ref[...] = m
        idx_ref[...] = gidx

    @pl.when(k > 0)
    def _():
        better = m < best_ref[...]
        idx_ref[...] = jnp.where(better, gidx, idx_ref[...])
        best_ref[...] = jnp.where(better, m, best_ref[...])


@functools.partial(jax.jit, static_argnames=("kk",))
def _dist_argmin(r, cb, a2, b2, kk):
    nt = _N // 2
    return pl.pallas_call(
        _dist_argmin_body,
        grid=(2, kk // _KB),
        in_specs=[
            pl.BlockSpec((nt, _D), lambda i, k: (i, 0)),
            pl.BlockSpec((_KB, _D), lambda i, k: (k, 0)),
            pl.BlockSpec((nt, 1), lambda i, k: (i, 0)),
            pl.BlockSpec((1, _KB), lambda i, k: (0, k)),
        ],
        out_specs=pl.BlockSpec((nt, 1), lambda i, k: (i, 0)),
        out_shape=jax.ShapeDtypeStruct((_N, 1), jnp.int32),
        scratch_shapes=[pltpu.VMEM((nt, 1), jnp.float32)],
        compiler_params=pltpu.CompilerParams(
            dimension_semantics=("parallel", "arbitrary"),
        ),
    )(r, cb, a2, b2)


def _sc_gather(cb, idx_row):
    mesh = plsc.VectorSubcoreMesh(core_axis_name="core", subcore_axis_name="subcore")
    dsub = _D // _SPLIT
    nrows = _N * _SPLIT
    cb_half = cb.reshape(-1, dsub)

    @pl.kernel(out_type=jax.ShapeDtypeStruct((nrows, dsub), jnp.float32), mesh=mesh)
    def kern(cb_hbm, i_hbm, o_hbm):
        def body(i_vmem, o_vmem):
            pltpu.sync_copy(cb_hbm.at[i_vmem.at[0]], o_vmem)

        pltpu.emit_pipeline(
            body,
            grid=(nrows // _GW,),
            in_specs=[pl.BlockSpec((1, _GW), lambda i: (0, i))],
            out_specs=[pl.BlockSpec((_GW, dsub), lambda i: (i, 0))],
            core_axis_name=("core", "subcore"),
            dimension_semantics=(pltpu.PARALLEL,),
        )(i_hbm, o_hbm)

    return kern(cb_half, idx_row).reshape(_N, _D)


def kernel(x, codebook_0, codebook_1, codebook_2, codebook_3):
    codebooks = [codebook_0, codebook_1, codebook_2, codebook_3]
    b, t, d = x.shape
    residual = x.reshape(-1, d)
    quantized = jnp.zeros_like(residual)
    all_indices = []
    total_commit = jnp.asarray(0.0, dtype=jnp.float32)
    for cb in codebooks:
        a2 = jnp.sum(residual * residual, axis=1, keepdims=True)
        b2 = jnp.sum(cb * cb, axis=1)[None, :]
        idx = _dist_argmin(residual, cb, a2, b2, cb.shape[0])
        half_idx = (idx * _SPLIT + jnp.arange(_SPLIT, dtype=jnp.int32)[None, :]).reshape(1, -1)
        q = _sc_gather(cb, half_idx)
        commit = jnp.mean((q - residual) ** 2) * 0.25
        total_commit = total_commit + commit
        q_st = residual + (q - residual)
        quantized = quantized + q_st
        residual = residual - q_st
        all_indices.append(idx.reshape(b, t))
    all_indices = jnp.stack(all_indices, axis=-1)
    return quantized.reshape(b, t, d), all_indices, total_commit


# speedup vs baseline: 1.5886x; 1.5886x over previous
import functools

import jax
import jax.numpy as jnp
from jax.experimental import pallas as pl
from jax.experimental.pallas import tpu as pltpu
from jax.experimental.pallas import tpu_sc as plsc

_N = 2048
_D = 768
_KB = 512
_NSC = 32
_GR = _N // _NSC


def _dist_argmin_body(r_ref, cb_ref, a2_ref, b2_ref, idx_ref, best_ref):
    k = pl.program_id(1)
    ab = jax.lax.dot_general(
        r_ref[...], cb_ref[...],
        dimension_numbers=(((1,), (1,)), ((), ())),
        preferred_element_type=jnp.float32,
    )
    s = a2_ref[...] + b2_ref[...]
    d2 = jnp.maximum(s - 2.0 * ab, 0.0)
    m = jnp.min(d2, axis=1, keepdims=True)
    j = jax.lax.broadcasted_iota(jnp.int32, d2.shape, 1)
    lidx = jnp.min(jnp.where(d2 == m, j, jnp.int32(2**30)), axis=1, keepdims=True)
    gidx = lidx + k * _KB

    @pl.when(k == 0)
    def _():
        best_ref[...] = m
        idx_ref[...] = gidx

    @pl.when(k > 0)
    def _():
        better = m < best_ref[...]
        idx_ref[...] = jnp.where(better, gidx, idx_ref[...])
        best_ref[...] = jnp.where(better, m, best_ref[...])


@functools.partial(jax.jit, static_argnames=("kk",))
def _dist_argmin(r, cb, a2, b2, kk):
    nt = _N // 2
    return pl.pallas_call(
        _dist_argmin_body,
        grid=(2, kk // _KB),
        in_specs=[
            pl.BlockSpec((nt, _D), lambda i, k: (i, 0)),
            pl.BlockSpec((_KB, _D), lambda i, k: (k, 0)),
            pl.BlockSpec((nt, 1), lambda i, k: (i, 0)),
            pl.BlockSpec((1, _KB), lambda i, k: (0, k)),
        ],
        out_specs=pl.BlockSpec((nt, 1), lambda i, k: (i, 0)),
        out_shape=jax.ShapeDtypeStruct((_N, 1), jnp.int32),
        scratch_shapes=[pltpu.VMEM((nt, 1), jnp.float32)],
        compiler_params=pltpu.CompilerParams(
            dimension_semantics=("parallel", "arbitrary"),
        ),
    )(r, cb, a2, b2)


def _sc_gather(cb, idx):
    mesh = plsc.VectorSubcoreMesh(core_axis_name="core", subcore_axis_name="subcore")

    @pl.kernel(
        out_type=jax.ShapeDtypeStruct((_N, _D), jnp.float32),
        mesh=mesh,
        scratch_types=[
            pltpu.VMEM((_GR, _D), jnp.float32),
            pltpu.VMEM((1, 128), jnp.int32),
        ],
    )
    def kern(cb_hbm, i_hbm, o_hbm, qbuf, ibuf):
        u = jax.lax.axis_index("core") * 16 + jax.lax.axis_index("subcore")
        pltpu.sync_copy(i_hbm.at[pl.ds(u // 2, 1)], ibuf)
        off = (u % 2) * _GR
        pltpu.sync_copy(cb_hbm.at[ibuf.at[0, pl.ds(off, _GR)]], qbuf)
        pltpu.sync_copy(qbuf, o_hbm.at[pl.ds(u * _GR, _GR)])

    return kern(cb, idx)


def kernel(x, codebook_0, codebook_1, codebook_2, codebook_3):
    codebooks = [codebook_0, codebook_1, codebook_2, codebook_3]
    b, t, d = x.shape
    x2d = x.reshape(-1, d)
    residual = x2d
    a2 = jnp.sum(residual * residual, axis=1, keepdims=True)
    all_indices = []
    commit_ssq = []
    for cb in codebooks:
        b2 = jnp.sum(cb * cb, axis=1)[None, :]
        idx = _dist_argmin(residual, cb, a2, b2, cb.shape[0])
        q = _sc_gather(cb, idx.reshape(16, 128))
        q_st = residual + (q - residual)
        residual = residual - q_st
        a2 = jnp.sum(residual * residual, axis=1, keepdims=True)
        commit_ssq.append(jnp.sum(a2))
        all_indices.append(idx.reshape(b, t))
    quantized = x2d - residual
    total_commit = (
        (commit_ssq[0] + commit_ssq[1] + commit_ssq[2] + commit_ssq[3])
        * (0.25 / (b * t * d))
    ).astype(jnp.float32)
    all_indices = jnp.stack(all_indices, axis=-1)
    return quantized.reshape(b, t, d), all_indices, total_commit
